# baseline (speedup 1.0000x reference)
import functools

import jax
import jax.numpy as jnp
from jax import lax
from jax.experimental import pallas as pl
from jax.experimental.pallas import tpu as pltpu
from jax.experimental.pallas import tpu_sc as plsc

B = 4096
L = 200
D = 64
NC = 2
NS = 16
NW = NC * NS
TOTAL = B * L
PER_TILE = TOTAL // NW
IDX_MINOR = 128
CHUNK = 512
IDX_ROWS = CHUNK // IDX_MINOR
STAGES = PER_TILE // CHUNK
TILE_IDX_ROWS = PER_TILE // IDX_MINOR


def _gather_body(idx_hbm, table_hbm, out_hbm, idx_v, rows_v, gat_sem):
    c = lax.axis_index("c")
    s = lax.axis_index("s")
    wid = s * NC + c
    idx_base = wid * TILE_IDX_ROWS
    row_base = wid * PER_TILE

    def stage(g, carry):
        pltpu.sync_copy(idx_hbm.at[pl.ds(idx_base + g * IDX_ROWS, IDX_ROWS)],
                        idx_v)
        copies = []
        for j in range(IDX_ROWS):
            copies.append(
                pltpu.async_copy(table_hbm.at[idx_v.at[j]],
                                 rows_v.at[pl.ds(j * IDX_MINOR, IDX_MINOR)],
                                 gat_sem))
        for cp in copies:
            cp.wait()
        pltpu.sync_copy(rows_v, out_hbm.at[pl.ds(row_base + g * CHUNK, CHUNK)])
        return carry

    lax.fori_loop(0, STAGES, stage, 0)


def _sc_gather(idx2d, table):
    mesh = plsc.VectorSubcoreMesh(core_axis_name="c", subcore_axis_name="s",
                                  num_cores=NC, num_subcores=NS)
    fn = pl.kernel(
        _gather_body,
        out_type=jax.ShapeDtypeStruct((TOTAL, D), jnp.float32),
        mesh=mesh,
        scratch_types=[
            pltpu.VMEM((IDX_ROWS, IDX_MINOR), jnp.int32),
            pltpu.VMEM((CHUNK, D), jnp.float32),
            pltpu.SemaphoreType.DMA,
        ],
        compiler_params=pltpu.CompilerParams(use_tc_tiling_on_sc=False),
    )
    return fn(idx2d, table)


PBLK = 2048


def _price_body(p_ref, w_ref, b_ref, o_ref):
    o_ref[...] = p_ref[...] * w_ref[...] + b_ref[...]


def _price_emb(price_flat, W, b):
    grid = (TOTAL // PBLK,)
    return pl.pallas_call(
        _price_body,
        grid=grid,
        in_specs=[
            pl.BlockSpec((PBLK, 1), lambda i: (i, 0)),
            pl.BlockSpec((1, D), lambda i: (0, 0)),
            pl.BlockSpec((1, D), lambda i: (0, 0)),
        ],
        out_specs=pl.BlockSpec((PBLK, D), lambda i: (i, 0)),
        out_shape=jax.ShapeDtypeStruct((TOTAL, D), jnp.float32),
    )(price_flat, W, b)


@jax.jit
def kernel(item_id, price, emb_table, W, b):
    idx2d = item_id.reshape(TOTAL // IDX_MINOR, IDX_MINOR)
    item_emb = _sc_gather(idx2d, emb_table).reshape(B, L, D)
    price_flat = price.reshape(TOTAL, 1)
    price_emb = _price_emb(price_flat, W, b.reshape(1, D)).reshape(B, L, D)
    return (item_emb, price_emb)
